# baseline (speedup 1.0000x reference)
import functools

import jax
import jax.numpy as jnp
from jax import lax
from jax.experimental import pallas as pl
from jax.experimental.pallas import tpu as pltpu
from jax.experimental.pallas import tpu_sc as plsc

B = 1024
S = 20
E = 128
RPS = 12
NC, NS = 2, 16
NW = NC * NS
SEGW = B // NW
NGROUP = E // 16

_mesh = plsc.VectorSubcoreMesh(core_axis_name="c", subcore_axis_name="s")


@functools.partial(
    pl.kernel,
    out_type=(jax.ShapeDtypeStruct((B, E), jnp.float32),
              jax.ShapeDtypeStruct((S * B, E), jnp.float32)),
    mesh=_mesh,
    scratch_types=[
        pltpu.VMEM((RPS * SEGW,), jnp.int32),
        pltpu.VMEM((S * RPS * SEGW,), jnp.int32),
        [pltpu.VMEM((SEGW, E), jnp.float32) for _ in range(2 * RPS)],
        [pltpu.VMEM((SEGW, E), jnp.float32) for _ in range(2)],
        [pltpu.SemaphoreType.DMA for _ in range(2)],
        [pltpu.SemaphoreType.DMA for _ in range(2)],
    ],
)
def _embed_kernel(obs_idx_hbm, act_idx_hbm, table_hbm, obs_hbm, act_hbm,
                  idx_o, idx_a, bufs, outb, gsem, osem):
    wid = lax.axis_index("s") * NC + lax.axis_index("c")
    col = wid * SEGW

    pltpu.sync_copy(obs_idx_hbm.at[wid], idx_o)
    pltpu.sync_copy(act_idx_hbm.at[wid], idx_a)

    def issue_act(g, p):
        for i in range(RPS):
            pltpu.async_copy(
                table_hbm.at[idx_a.at[pl.ds((g * RPS + i) * SEGW, SEGW)]],
                bufs[RPS * p + i], gsem[p])

    def drain_g(p):
        for i in range(RPS):
            pltpu.make_async_copy(table_hbm.at[pl.ds(0, SEGW)],
                                  bufs[RPS * p + i], gsem[p]).wait()

    def wait_out(ob):
        pltpu.make_async_copy(table_hbm.at[pl.ds(0, SEGW)], outb[ob],
                              osem[ob]).wait()

    def reduce_store(p, ob, dst_ref, dst_row):
        @plsc.parallel_loop(0, SEGW, step=1, unroll=2)
        def _body(b):
            for gr in range(NGROUP):
                sl = pl.ds(gr * 16, 16)
                acc = bufs[RPS * p][b, sl]
                for i in range(1, RPS):
                    acc = acc + bufs[RPS * p + i][b, sl]
                outb[ob][b, sl] = acc * (1.0 / RPS)

        pltpu.async_copy(outb[ob], dst_ref.at[pl.ds(dst_row, SEGW)], osem[ob])

    for i in range(RPS):
        pltpu.async_copy(table_hbm.at[idx_o.at[pl.ds(i * SEGW, SEGW)]],
                         bufs[i], gsem[0])
    issue_act(0, 1)
    drain_g(0)
    reduce_store(0, 0, obs_hbm, col)

    def pair_body(k, _):
        g = 2 * k
        issue_act(g + 1, 0)
        drain_g(1)

        @pl.when(k > 0)
        def _w1():
            wait_out(1)

        reduce_store(1, 1, act_hbm, g * B + col)

        @pl.when(g + 2 < S)
        def _i2():
            issue_act(g + 2, 1)

        drain_g(0)
        wait_out(0)
        reduce_store(0, 0, act_hbm, (g + 1) * B + col)
        return 0

    lax.fori_loop(0, S // 2, pair_body, 0)
    wait_out(0)
    wait_out(1)


def kernel(sub_index, derived_sub_indices, action_mask, table):
    obs_t = jnp.transpose(sub_index.astype(jnp.int32),
                          (1, 3, 2, 0)).reshape(RPS, NW, SEGW)
    obs_w = jnp.transpose(obs_t, (1, 0, 2)).reshape(NW, RPS * SEGW)
    act_t = jnp.transpose(derived_sub_indices.astype(jnp.int32),
                          (1, 3, 2, 0)).reshape(S * RPS, NW, SEGW)
    act_w = jnp.transpose(act_t, (1, 0, 2)).reshape(NW, S * RPS * SEGW)
    obs, act = _embed_kernel(obs_w, act_w, table)
    obs = obs.reshape(B, 1, E)
    act = act.reshape(S, B, E).transpose(1, 0, 2)
    return (obs, act, action_mask)


# speedup vs baseline: 1.0054x; 1.0054x over previous
import functools

import jax
import jax.numpy as jnp
from jax import lax
from jax.experimental import pallas as pl
from jax.experimental.pallas import tpu as pltpu
from jax.experimental.pallas import tpu_sc as plsc

B = 1024
S = 20
E = 128
RPS = 12
NC, NS = 2, 16
NW = NC * NS
SEGW = B // NW
NGROUP = E // 16

_mesh = plsc.VectorSubcoreMesh(core_axis_name="c", subcore_axis_name="s")


@functools.partial(
    pl.kernel,
    out_type=(jax.ShapeDtypeStruct((B, E), jnp.float32),
              jax.ShapeDtypeStruct((S * B, E), jnp.float32)),
    mesh=_mesh,
    scratch_types=[
        pltpu.VMEM((RPS * SEGW,), jnp.int32),
        pltpu.VMEM((S * RPS * SEGW,), jnp.int32),
        [pltpu.VMEM((SEGW, E), jnp.float32) for _ in range(2 * RPS)],
        [pltpu.VMEM((SEGW, E), jnp.float32) for _ in range(2)],
        [pltpu.SemaphoreType.DMA for _ in range(2)],
        [pltpu.SemaphoreType.DMA for _ in range(2)],
    ],
)
def _embed_kernel(obs_idx_hbm, act_idx_hbm, table_hbm, obs_hbm, act_hbm,
                  idx_o, idx_a, bufs, outb, gsem, osem):
    wid = lax.axis_index("s") * NC + lax.axis_index("c")
    col = wid * SEGW

    pltpu.sync_copy(obs_idx_hbm.at[wid], idx_o)
    pltpu.sync_copy(act_idx_hbm.at[wid], idx_a)

    def issue_act(g, p):
        for i in range(RPS):
            pltpu.async_copy(
                table_hbm.at[idx_a.at[pl.ds((g * RPS + i) * SEGW, SEGW)]],
                bufs[RPS * p + i], gsem[p])

    def drain_g(p):
        for i in range(RPS):
            pltpu.make_async_copy(table_hbm.at[pl.ds(0, SEGW)],
                                  bufs[RPS * p + i], gsem[p]).wait()

    def wait_out(ob):
        pltpu.make_async_copy(table_hbm.at[pl.ds(0, SEGW)], outb[ob],
                              osem[ob]).wait()

    def reduce_store(p, ob, dst_ref, dst_row):
        def body(h, _):
            for u in range(2):
                b = 2 * h + u
                for gr in range(NGROUP):
                    sl = pl.ds(gr * 16, 16)
                    acc = bufs[RPS * p][b, sl]
                    for i in range(1, RPS):
                        acc = acc + bufs[RPS * p + i][b, sl]
                    outb[ob][b, sl] = acc * (1.0 / RPS)
            return 0

        lax.fori_loop(0, SEGW // 2, body, 0)
        pltpu.async_copy(outb[ob], dst_ref.at[pl.ds(dst_row, SEGW)], osem[ob])

    for i in range(RPS):
        pltpu.async_copy(table_hbm.at[idx_o.at[pl.ds(i * SEGW, SEGW)]],
                         bufs[i], gsem[0])
    issue_act(0, 1)
    drain_g(0)
    reduce_store(0, 0, obs_hbm, col)

    def pair_body(k, _):
        g = 2 * k
        issue_act(g + 1, 0)
        drain_g(1)

        @pl.when(k > 0)
        def _w1():
            wait_out(1)

        reduce_store(1, 1, act_hbm, g * B + col)

        @pl.when(g + 2 < S)
        def _i2():
            issue_act(g + 2, 1)

        drain_g(0)
        wait_out(0)
        reduce_store(0, 0, act_hbm, (g + 1) * B + col)
        return 0

    lax.fori_loop(0, S // 2, pair_body, 0)
    wait_out(0)
    wait_out(1)


def kernel(sub_index, derived_sub_indices, action_mask, table):
    obs_t = jnp.transpose(sub_index.astype(jnp.int32),
                          (1, 3, 2, 0)).reshape(RPS, NW, SEGW)
    obs_w = jnp.transpose(obs_t, (1, 0, 2)).reshape(NW, RPS * SEGW)
    act_t = jnp.transpose(derived_sub_indices.astype(jnp.int32),
                          (1, 3, 2, 0)).reshape(S * RPS, NW, SEGW)
    act_w = jnp.transpose(act_t, (1, 0, 2)).reshape(NW, S * RPS * SEGW)
    obs, act = _embed_kernel(obs_w, act_w, table)
    obs = obs.reshape(B, 1, E)
    act = act.reshape(S, B, E).transpose(1, 0, 2)
    return (obs, act, action_mask)


# speedup vs baseline: 1.0558x; 1.0501x over previous
import functools

import jax
import jax.numpy as jnp
from jax import lax
from jax.experimental import pallas as pl
from jax.experimental.pallas import tpu as pltpu
from jax.experimental.pallas import tpu_sc as plsc

B = 1024
S = 20
E = 128
RPS = 12
NC, NS = 2, 16
NW = NC * NS
SEGW = B // NW
NGROUP = E // 16

_mesh = plsc.VectorSubcoreMesh(core_axis_name="c", subcore_axis_name="s")


@functools.partial(
    pl.kernel,
    out_type=(jax.ShapeDtypeStruct((B, E), jnp.float32),
              jax.ShapeDtypeStruct((S * B, E), jnp.float32)),
    mesh=_mesh,
    scratch_types=[
        pltpu.VMEM((RPS * SEGW,), jnp.int32),
        pltpu.VMEM((S * RPS * SEGW,), jnp.int32),
        [pltpu.VMEM((SEGW, E), jnp.float32) for _ in range(2 * RPS)],
        [pltpu.VMEM((SEGW, E), jnp.float32) for _ in range(2)],
        [pltpu.SemaphoreType.DMA for _ in range(2)],
        [pltpu.SemaphoreType.DMA for _ in range(2)],
    ],
)
def _embed_kernel(obs_idx_hbm, act_idx_hbm, table_hbm, obs_hbm, act_hbm,
                  idx_o, idx_a, bufs, outb, gsem, osem):
    wid = lax.axis_index("s") * NC + lax.axis_index("c")
    col = wid * SEGW

    pltpu.sync_copy(obs_idx_hbm.at[wid], idx_o)
    pltpu.sync_copy(act_idx_hbm.at[wid], idx_a)

    def issue_act(g, p):
        for i in range(RPS):
            pltpu.async_copy(
                table_hbm.at[idx_a.at[pl.ds((g * RPS + i) * SEGW, SEGW)]],
                bufs[RPS * p + i], gsem[p])

    def drain_g(p):
        for i in range(RPS):
            pltpu.make_async_copy(table_hbm.at[pl.ds(0, SEGW)],
                                  bufs[RPS * p + i], gsem[p]).wait()

    def wait_out(ob):
        pltpu.make_async_copy(table_hbm.at[pl.ds(0, SEGW)], outb[ob],
                              osem[ob]).wait()

    def reduce_store(p, ob, dst_ref, dst_row):
        def body(b, _):
            for gr in range(NGROUP):
                sl = pl.ds(gr * 16, 16)
                acc = bufs[RPS * p][b, sl]
                for i in range(1, RPS):
                    acc = acc + bufs[RPS * p + i][b, sl]
                outb[ob][b, sl] = acc * (1.0 / RPS)
            return 0

        lax.fori_loop(0, SEGW, body, 0)
        pltpu.async_copy(outb[ob], dst_ref.at[pl.ds(dst_row, SEGW)], osem[ob])

    for i in range(RPS):
        pltpu.async_copy(table_hbm.at[idx_o.at[pl.ds(i * SEGW, SEGW)]],
                         bufs[i], gsem[0])
    issue_act(0, 1)
    drain_g(0)
    reduce_store(0, 0, obs_hbm, col)

    def pair_body(k, _):
        g = 2 * k
        issue_act(g + 1, 0)
        drain_g(1)

        @pl.when(k > 0)
        def _w1():
            wait_out(1)

        reduce_store(1, 1, act_hbm, g * B + col)

        @pl.when(g + 2 < S)
        def _i2():
            issue_act(g + 2, 1)

        drain_g(0)
        wait_out(0)
        reduce_store(0, 0, act_hbm, (g + 1) * B + col)
        return 0

    lax.fori_loop(0, S // 2, pair_body, 0)
    wait_out(0)
    wait_out(1)


def kernel(sub_index, derived_sub_indices, action_mask, table):
    obs_t = jnp.transpose(sub_index.astype(jnp.int32),
                          (1, 3, 2, 0)).reshape(RPS, NW, SEGW)
    obs_w = jnp.transpose(obs_t, (1, 0, 2)).reshape(NW, RPS * SEGW)
    act_t = jnp.transpose(derived_sub_indices.astype(jnp.int32),
                          (1, 3, 2, 0)).reshape(S * RPS, NW, SEGW)
    act_w = jnp.transpose(act_t, (1, 0, 2)).reshape(NW, S * RPS * SEGW)
    obs, act = _embed_kernel(obs_w, act_w, table)
    obs = obs.reshape(B, 1, E)
    act = act.reshape(S, B, E).transpose(1, 0, 2)
    return (obs, act, action_mask)


# speedup vs baseline: 1.0811x; 1.0239x over previous
import functools

import jax
import jax.numpy as jnp
from jax import lax
from jax.experimental import pallas as pl
from jax.experimental.pallas import tpu as pltpu
from jax.experimental.pallas import tpu_sc as plsc

B = 1024
S = 20
E = 128
RPS = 12
NC, NS = 2, 16
NW = NC * NS
SEGW = B // NW
NGROUP = E // 16

_mesh = plsc.VectorSubcoreMesh(core_axis_name="c", subcore_axis_name="s")


@functools.partial(
    pl.kernel,
    out_type=(jax.ShapeDtypeStruct((B, E), jnp.float32),
              jax.ShapeDtypeStruct((S * B, E), jnp.float32)),
    mesh=_mesh,
    scratch_types=[
        pltpu.VMEM((RPS * SEGW,), jnp.int32),
        pltpu.VMEM((S * RPS * SEGW,), jnp.int32),
        [pltpu.VMEM((SEGW, E), jnp.float32) for _ in range(2 * RPS)],
        [pltpu.VMEM((SEGW, E), jnp.float32) for _ in range(2)],
        [pltpu.SemaphoreType.DMA for _ in range(2)],
        [pltpu.SemaphoreType.DMA for _ in range(2)],
        pltpu.SemaphoreType.DMA,
    ],
)
def _embed_kernel(obs_idx_hbm, act_idx_hbm, table_hbm, obs_hbm, act_hbm,
                  idx_o, idx_a, bufs, outb, gsem, osem, isem):
    wid = lax.axis_index("s") * NC + lax.axis_index("c")
    col = wid * SEGW

    pltpu.sync_copy(obs_idx_hbm.at[wid], idx_o)
    pltpu.async_copy(act_idx_hbm.at[wid], idx_a, isem)

    def issue_act(g, p):
        for i in range(RPS):
            pltpu.async_copy(
                table_hbm.at[idx_a.at[pl.ds((g * RPS + i) * SEGW, SEGW)]],
                bufs[RPS * p + i], gsem[p])

    def drain_g(p):
        for i in range(RPS):
            pltpu.make_async_copy(table_hbm.at[pl.ds(0, SEGW)],
                                  bufs[RPS * p + i], gsem[p]).wait()

    def wait_out(ob):
        pltpu.make_async_copy(table_hbm.at[pl.ds(0, SEGW)], outb[ob],
                              osem[ob]).wait()

    def reduce_store(p, ob, dst_ref, dst_row):
        def body(b, _):
            for gr in range(NGROUP):
                sl = pl.ds(gr * 16, 16)
                acc = bufs[RPS * p][b, sl]
                for i in range(1, RPS):
                    acc = acc + bufs[RPS * p + i][b, sl]
                outb[ob][b, sl] = acc * (1.0 / RPS)
            return 0

        lax.fori_loop(0, SEGW, body, 0)
        pltpu.async_copy(outb[ob], dst_ref.at[pl.ds(dst_row, SEGW)], osem[ob])

    for i in range(RPS):
        pltpu.async_copy(table_hbm.at[idx_o.at[pl.ds(i * SEGW, SEGW)]],
                         bufs[i], gsem[0])
    pltpu.make_async_copy(act_idx_hbm.at[wid], idx_a, isem).wait()
    issue_act(0, 1)
    drain_g(0)
    reduce_store(0, 0, obs_hbm, col)

    def pair_body(k, _):
        g = 2 * k
        issue_act(g + 1, 0)
        drain_g(1)

        @pl.when(k > 0)
        def _w1():
            wait_out(1)

        reduce_store(1, 1, act_hbm, g * B + col)

        @pl.when(g + 2 < S)
        def _i2():
            issue_act(g + 2, 1)

        drain_g(0)
        wait_out(0)
        reduce_store(0, 0, act_hbm, (g + 1) * B + col)
        return 0

    lax.fori_loop(0, S // 2, pair_body, 0)
    wait_out(0)
    wait_out(1)


def kernel(sub_index, derived_sub_indices, action_mask, table):
    obs_t = jnp.transpose(sub_index.astype(jnp.int32),
                          (1, 3, 2, 0)).reshape(RPS, NW, SEGW)
    obs_w = jnp.transpose(obs_t, (1, 0, 2)).reshape(NW, RPS * SEGW)
    act_t = jnp.transpose(derived_sub_indices.astype(jnp.int32),
                          (1, 3, 2, 0)).reshape(S * RPS, NW, SEGW)
    act_w = jnp.transpose(act_t, (1, 0, 2)).reshape(NW, S * RPS * SEGW)
    obs, act = _embed_kernel(obs_w, act_w, table)
    obs = obs.reshape(B, 1, E)
    act = act.reshape(S, B, E).transpose(1, 0, 2)
    return (obs, act, action_mask)


# speedup vs baseline: 1.1912x; 1.1019x over previous
import functools

import jax
import jax.numpy as jnp
from jax import lax
from jax.experimental import pallas as pl
from jax.experimental.pallas import tpu as pltpu
from jax.experimental.pallas import tpu_sc as plsc

B = 1024
S = 20
E = 128
RPS = 12
NC, NS = 2, 16
NW = NC * NS
SEGW = B // NW
NGROUP = E // 16

_mesh = plsc.VectorSubcoreMesh(core_axis_name="c", subcore_axis_name="s")


@functools.partial(
    pl.kernel,
    out_type=(jax.ShapeDtypeStruct((B, E), jnp.float32),
              jax.ShapeDtypeStruct((S * B, E), jnp.float32)),
    mesh=_mesh,
    scratch_types=[
        pltpu.VMEM((RPS * SEGW,), jnp.int32),
        pltpu.VMEM((S * RPS * SEGW,), jnp.int32),
        [pltpu.VMEM((SEGW, E), jnp.float32) for _ in range(2 * RPS)],
        [pltpu.VMEM((SEGW, E), jnp.float32) for _ in range(2)],
        [pltpu.SemaphoreType.DMA for _ in range(2)],
        [pltpu.SemaphoreType.DMA for _ in range(2)],
        pltpu.SemaphoreType.DMA,
    ],
)
def _embed_kernel(obs_idx_hbm, act_idx_hbm, table_hbm, obs_hbm, act_hbm,
                  idx_o, idx_a, bufs, outb, gsem, osem, isem):
    wid = lax.axis_index("s") * NC + lax.axis_index("c")
    col = wid * SEGW

    for i in range(RPS):
        pltpu.async_copy(obs_idx_hbm.at[pl.ds(i * B + col, SEGW)],
                         idx_o.at[pl.ds(i * SEGW, SEGW)], isem)
    pltpu.make_async_copy(obs_idx_hbm.at[pl.ds(0, RPS * SEGW)], idx_o,
                          isem).wait()

    def issue_act(g, p):
        for i in range(RPS):
            pltpu.async_copy(
                table_hbm.at[idx_a.at[pl.ds((g * RPS + i) * SEGW, SEGW)]],
                bufs[RPS * p + i], gsem[p])

    def drain_g(p):
        for i in range(RPS):
            pltpu.make_async_copy(table_hbm.at[pl.ds(0, SEGW)],
                                  bufs[RPS * p + i], gsem[p]).wait()

    def wait_out(ob):
        pltpu.make_async_copy(table_hbm.at[pl.ds(0, SEGW)], outb[ob],
                              osem[ob]).wait()

    def reduce_store(p, ob, dst_ref, dst_row):
        def body(b, _):
            for gr in range(NGROUP):
                sl = pl.ds(gr * 16, 16)
                acc = bufs[RPS * p][b, sl]
                for i in range(1, RPS):
                    acc = acc + bufs[RPS * p + i][b, sl]
                outb[ob][b, sl] = acc * (1.0 / RPS)
            return 0

        lax.fori_loop(0, SEGW, body, 0)
        pltpu.async_copy(outb[ob], dst_ref.at[pl.ds(dst_row, SEGW)], osem[ob])

    for i in range(RPS):
        pltpu.async_copy(table_hbm.at[idx_o.at[pl.ds(i * SEGW, SEGW)]],
                         bufs[i], gsem[0])
    def stage_act(i, _):
        pltpu.async_copy(act_idx_hbm.at[pl.ds(i * B + col, SEGW)],
                         idx_a.at[pl.ds(i * SEGW, SEGW)], isem)
        return 0

    lax.fori_loop(0, S * RPS, stage_act, 0)
    pltpu.make_async_copy(act_idx_hbm.at[pl.ds(0, S * RPS * SEGW)], idx_a,
                          isem).wait()
    issue_act(0, 1)
    drain_g(0)
    reduce_store(0, 0, obs_hbm, col)

    def pair_body(k, _):
        g = 2 * k
        issue_act(g + 1, 0)
        drain_g(1)

        @pl.when(k > 0)
        def _w1():
            wait_out(1)

        reduce_store(1, 1, act_hbm, g * B + col)

        @pl.when(g + 2 < S)
        def _i2():
            issue_act(g + 2, 1)

        drain_g(0)
        wait_out(0)
        reduce_store(0, 0, act_hbm, (g + 1) * B + col)
        return 0

    lax.fori_loop(0, S // 2, pair_body, 0)
    wait_out(0)
    wait_out(1)


def kernel(sub_index, derived_sub_indices, action_mask, table):
    obs_w = jnp.transpose(sub_index.astype(jnp.int32),
                          (1, 3, 2, 0)).reshape(RPS * B)
    act_w = jnp.transpose(derived_sub_indices.astype(jnp.int32),
                          (1, 3, 2, 0)).reshape(S * RPS * B)
    obs, act = _embed_kernel(obs_w, act_w, table)
    obs = obs.reshape(B, 1, E)
    act = act.reshape(S, B, E).transpose(1, 0, 2)
    return (obs, act, action_mask)
